# baseline (speedup 1.0000x reference)
import functools

import jax
import jax.numpy as jnp
import numpy as np
from jax.experimental import pallas as pl
from jax.experimental.pallas import tpu as pltpu

R_MAX = 5.0
NUM_RBF = 32
DEGREES = (1, 2)
F = 128
H = 4
L = 2
AVG_NEI = 32.0
M_TOT = 8
ND = 2
EPS = 1e-8
SLICES = ((0, 3), (3, 8))


def _sph(u):
    x, y, z = u[:, 0], u[:, 1], u[:, 2]
    c1 = np.sqrt(3.0 / (4.0 * np.pi))
    y1 = jnp.stack([c1 * y, c1 * z, c1 * x], axis=1)
    c2a = 0.5 * np.sqrt(15.0 / np.pi)
    c2b = 0.25 * np.sqrt(5.0 / np.pi)
    c2c = 0.25 * np.sqrt(15.0 / np.pi)
    y2 = jnp.stack([c2a * x * y, c2a * y * z, c2b * (3.0 * z * z - 1.0), c2a * x * z, c2c * (x * x - y * y)], axis=1)
    return jnp.concatenate([y1, y2], axis=1)


def _rbf(r):
    centers = jnp.linspace(0.0, R_MAX, NUM_RBF)
    gamma = (NUM_RBF / R_MAX) ** 2
    return jnp.exp(-gamma * (r[:, None] - centers[None, :]) ** 2)


def _cutoff(r):
    return jnp.where(r < R_MAX, 0.5 * (jnp.cos(np.pi * r / R_MAX) + 1.0), 0.0)


def _readout_body(x_ref, wh1_ref, bh1_ref, wh2_ref, bh2_ref, out_ref):
    i = pl.program_id(0)
    h = jax.nn.silu(x_ref[...] @ wh1_ref[...] + bh1_ref[...])
    e = h @ wh2_ref[...] + bh2_ref[...]
    s = jnp.sum(e).reshape(1, 1)

    @pl.when(i == 0)
    def _():
        out_ref[...] = jnp.zeros_like(out_ref)

    out_ref[...] += s


def _readout(x, Wh1, bh1, Wh2, bh2):
    n = x.shape[0]
    blk = 2000
    grid = n // blk
    out = pl.pallas_call(
        _readout_body,
        grid=(grid,),
        in_specs=[
            pl.BlockSpec((blk, F), lambda i: (i, 0)),
            pl.BlockSpec((F, F), lambda i: (0, 0)),
            pl.BlockSpec((F,), lambda i: (0,)),
            pl.BlockSpec((F, 1), lambda i: (0, 0)),
            pl.BlockSpec((1,), lambda i: (0,)),
        ],
        out_specs=pl.BlockSpec((1, 1), lambda i: (0, 0)),
        out_shape=jax.ShapeDtypeStruct((1, 1), jnp.float32),
    )(x, Wh1, bh1, Wh2, bh2)
    return out.reshape(1)


def kernel(positions, atomic_numbers, edge_index, emb, Wq, Wk, Wv, Wr, Wqe, Wke, Wre, W1, b1, W2, b2, Wh1, bh1, Wh2, bh2):
    src = edge_index[0]
    dst = edge_index[1]
    n = positions.shape[0]
    vec = positions[src] - positions[dst]
    r = jnp.sqrt(jnp.sum(vec * vec, axis=1) + EPS)
    u = vec / r[:, None]
    rbf = _rbf(r)
    cut = _cutoff(r)
    sph = _sph(u)
    x = jnp.take(emb, atomic_numbers, axis=0)
    ev = jnp.zeros((n, M_TOT), dtype=positions.dtype)
    dh = F // H
    de = F // ND
    for l in range(L):
        q = x @ Wq[l]
        k = x @ Wk[l]
        v = x @ Wv[l]
        w = rbf @ Wr[l]
        alpha = jnp.sum((q[dst] * w * k[src]).reshape(-1, H, dh), axis=-1) / np.sqrt(dh)
        alpha = alpha * cut[:, None]
        m = jnp.repeat(alpha, dh, axis=1) * v[src]
        x = x + jax.ops.segment_sum(m, dst, num_segments=n) / AVG_NEI
        qe = x @ Wqe[l]
        ke = x @ Wke[l]
        we = rbf @ Wre[l]
        ae = jnp.sum((qe[dst] * we * ke[src]).reshape(-1, ND, de), axis=-1) / np.sqrt(de)
        ae = ae * cut[:, None]
        rep = jnp.concatenate([jnp.repeat(ae[:, i:i + 1], 2 * DEGREES[i] + 1, axis=1) for i in range(ND)], axis=1)
        ev = ev + jax.ops.segment_sum(rep * sph, dst, num_segments=n) / AVG_NEI
        d = jnp.concatenate([jnp.sum(ev[:, a:b] ** 2, axis=1, keepdims=True) for (a, b) in SLICES], axis=1)
        y = jnp.concatenate([x, d], axis=1)
        h = jax.nn.silu(y @ W1[l] + b1[l])
        out = h @ W2[l] + b2[l]
        x = x + out[:, :F]
        dsc = out[:, F:]
        evrep = jnp.concatenate([jnp.repeat(dsc[:, i:i + 1], 2 * DEGREES[i] + 1, axis=1) for i in range(ND)], axis=1)
        ev = ev + evrep * ev
    return _readout(x, Wh1, bh1, Wh2, bh2)


# speedup vs baseline: 3.2036x; 3.2036x over previous
import functools

import jax
import jax.numpy as jnp
import numpy as np
from jax import lax
from jax.experimental import pallas as pl
from jax.experimental.pallas import tpu as pltpu
from jax.experimental.pallas import tpu_sc as plsc

R_MAX = 5.0
NUM_RBF = 32
F = 128
H = 4
AVG_NEI = 32.0
ND = 2
EPS = 1e-8

NC = 2
NS = 16
NW = NC * NS

@functools.cache
def _sc_mesh():
    return plsc.VectorSubcoreMesh(core_axis_name="c", subcore_axis_name="s",
                                  num_cores=NC, num_subcores=NS)

_MH = (np.arange(F)[:, None] // 32 == np.arange(H)[None, :]).astype(np.float32) / np.sqrt(32.0)
_MHT = (np.arange(F)[None, :] // 32 == np.arange(H)[:, None]).astype(np.float32)
_ME = (np.arange(F)[:, None] // 64 == np.arange(ND)[None, :]).astype(np.float32) / np.sqrt(64.0)
_REP2 = np.zeros((ND, 8), np.float32)
_REP2[0, 0:3] = 1.0
_REP2[1, 3:8] = 1.0
_REP2T = _REP2.T.copy()


def _pick_chunk(per_w, d, budget_bytes=204800):
    c = 8
    for cand in range(8, per_w + 1, 8):
        if per_w % cand == 0 and cand * d * 4 <= budget_bytes:
            c = cand
    return c


def _sc_gather(table, idx, d):
    e = idx.shape[0]
    assert e % NW == 0
    per_w = e // NW
    c = _pick_chunk(per_w, d)
    n_iter = per_w // c

    @functools.partial(
        pl.kernel,
        out_type=jax.ShapeDtypeStruct((e, d), jnp.float32),
        mesh=_sc_mesh(),
        scratch_types=[
            pltpu.VMEM((c,), jnp.int32),
            pltpu.VMEM((c, d), jnp.float32),
            pltpu.SemaphoreType.DMA,
        ],
    )
    def g(table_hbm, idx_hbm, out_hbm, idx_v, rows_v, sem):
        wid = lax.axis_index("s") * NC + lax.axis_index("c")
        base = wid * per_w

        def body(j, carry):
            off = base + j * c
            pltpu.sync_copy(idx_hbm.at[pl.ds(off, c)], idx_v)
            pltpu.async_copy(table_hbm.at[idx_v], rows_v, sem).wait()
            pltpu.sync_copy(rows_v, out_hbm.at[pl.ds(off, c)])
            return carry

        lax.fori_loop(0, n_iter, body, 0)

    return g(table, idx)


def _sc_scatter_add(rows, idx, n, d):
    e = idx.shape[0]
    assert e % NW == 0
    per_w = e // NW
    c = _pick_chunk(per_w, d, budget_bytes=102400)
    n_iter = per_w // c
    assert n % c == 0
    n_out_chunks = n // c
    out_iters = (n_out_chunks + NS - 1) // NS
    assert n % 16 == 0
    n_zchunks = n // 16
    z_iters = (n_zchunks + NS - 1) // NS

    @functools.partial(
        pl.kernel,
        out_type=jax.ShapeDtypeStruct((NC, n, d), jnp.float32),
        mesh=_sc_mesh(),
        scratch_types=[
            pltpu.VMEM((c,), jnp.int32),
            pltpu.VMEM((c, d), jnp.float32),
            pltpu.VMEM((16, d), jnp.float32),
            pltpu.VMEM_SHARED((n, d), jnp.float32),
            pltpu.SemaphoreType.DMA,
        ],
    )
    def s(rows_hbm, idx_hbm, out_hbm, idx_v, rows_v, zrow_v, acc, sem):
        cid = lax.axis_index("c")
        sid = lax.axis_index("s")
        wid = sid * NC + cid
        base = wid * per_w

        def zb(i, carry):
            r = i // (d // 16)
            col = (i % (d // 16)) * 16
            zrow_v[r, pl.ds(col, 16)] = jnp.zeros((16,), jnp.float32)
            return carry

        lax.fori_loop(0, 16 * (d // 16), zb, 0)

        def zc(i, carry):
            k = sid + NS * i

            @pl.when(k < n_zchunks)
            def _():
                pltpu.sync_copy(zrow_v, acc.at[pl.ds(k * 16, 16)])

            return carry

        lax.fori_loop(0, z_iters, zc, 0)
        plsc.subcore_barrier()

        def body(j, carry):
            off = base + j * c
            pltpu.sync_copy(idx_hbm.at[pl.ds(off, c)], idx_v)
            pltpu.sync_copy(rows_hbm.at[pl.ds(off, c)], rows_v)
            pltpu.sync_copy(rows_v, acc.at[idx_v], add=True)
            return carry

        lax.fori_loop(0, n_iter, body, 0)
        plsc.subcore_barrier()

        def oc(i, carry):
            k = sid + NS * i

            @pl.when(k < n_out_chunks)
            def _():
                pltpu.sync_copy(acc.at[pl.ds(k * c, c)], rows_v)
                pltpu.sync_copy(rows_v, out_hbm.at[cid].at[pl.ds(k * c, c)])

            return carry

        lax.fori_loop(0, out_iters, oc, 0)

    return s(rows, idx)


def _sc_edge_vec(pos4, src, dst):
    n4 = pos4.shape[0] * 4
    posf = pos4.reshape(n4)
    e = src.shape[0]
    assert e % NW == 0
    per_w = e // NW
    c = 400
    assert per_w % c == 0 and c % 16 == 0
    n_iter = per_w // c
    n_grp = c // 16

    @functools.partial(
        pl.kernel,
        out_type=jax.ShapeDtypeStruct((e * 4,), jnp.float32),
        mesh=_sc_mesh(),
        compiler_params=pltpu.CompilerParams(needs_layout_passes=False),
        scratch_types=[
            pltpu.VMEM((n4,), jnp.float32),
            pltpu.VMEM((c,), jnp.int32),
            pltpu.VMEM((c,), jnp.int32),
            pltpu.VMEM((c * 4,), jnp.float32),
        ],
    )
    def g(pos_hbm, src_hbm, dst_hbm, out_hbm, tab_v, src_v, dst_v, out_v):
        wid = lax.axis_index("s") * NC + lax.axis_index("c")
        base = wid * per_w
        pltpu.sync_copy(pos_hbm, tab_v)
        zero16 = jnp.zeros((16,), jnp.float32)

        def body(j, carry):
            off = base + j * c
            pltpu.sync_copy(src_hbm.at[pl.ds(off, c)], src_v)
            pltpu.sync_copy(dst_hbm.at[pl.ds(off, c)], dst_v)

            def grp(gi, carry2):
                rows = jnp.arange(16, dtype=jnp.int32) + gi * 16
                sv = src_v[pl.ds(gi * 16, 16)] * 4
                dv = dst_v[pl.ds(gi * 16, 16)] * 4
                rv = rows * 4
                for comp in range(4):
                    if comp < 3:
                        a = plsc.load_gather(tab_v, [sv + comp])
                        b = plsc.load_gather(tab_v, [dv + comp])
                        val = a - b
                    else:
                        val = zero16
                    plsc.store_scatter(out_v, [rv + comp], val)
                return carry2

            lax.fori_loop(0, n_grp, grp, 0)
            pltpu.sync_copy(out_v, out_hbm.at[pl.ds(off * 4, c * 4)])
            return carry

        lax.fori_loop(0, n_iter, body, 0)

    return g(posf, src, dst).reshape(e, 4)


def _geom_body(dv_ref, cen_ref, rbfc_ref, sph_ref):
    dvec = dv_ref[...]
    r2 = jnp.sum(dvec * dvec, axis=1, keepdims=True)
    r = jnp.sqrt(r2 + EPS)
    inv = 1.0 / r
    ux = dvec[:, 0:1] * inv
    uy = dvec[:, 1:2] * inv
    uz = dvec[:, 2:3] * inv
    centers = cen_ref[...]
    gamma = (NUM_RBF / R_MAX) ** 2
    rbf = jnp.exp(-gamma * (r - centers) ** 2)
    cut = jnp.where(r < R_MAX, 0.5 * (jnp.cos(np.pi * r / R_MAX) + 1.0), 0.0)
    rbfc_ref[...] = rbf * cut
    c1 = np.sqrt(3.0 / (4.0 * np.pi))
    c2a = 0.5 * np.sqrt(15.0 / np.pi)
    c2b = 0.25 * np.sqrt(5.0 / np.pi)
    c2c = 0.25 * np.sqrt(15.0 / np.pi)
    sph_ref[...] = jnp.concatenate([
        c1 * uy, c1 * uz, c1 * ux,
        c2a * ux * uy, c2a * uy * uz, c2b * (3.0 * uz * uz - 1.0),
        c2a * ux * uz, c2c * (ux * ux - uy * uy),
    ], axis=1)


def _geom(dvec, blk=2000):
    e = dvec.shape[0]
    grid = e // blk
    return pl.pallas_call(
        _geom_body,
        grid=(grid,),
        in_specs=[
            pl.BlockSpec((blk, 4), lambda i: (i, 0)),
            pl.BlockSpec((1, NUM_RBF), lambda i: (0, 0)),
        ],
        out_specs=[
            pl.BlockSpec((blk, NUM_RBF), lambda i: (i, 0)),
            pl.BlockSpec((blk, 8), lambda i: (i, 0)),
        ],
        out_shape=[
            jax.ShapeDtypeStruct((e, NUM_RBF), jnp.float32),
            jax.ShapeDtypeStruct((e, 8), jnp.float32),
        ],
    )(dvec, jnp.asarray(np.linspace(0.0, R_MAX, NUM_RBF, dtype=np.float32).reshape(1, NUM_RBF)))


def _qkv_body(x_ref, wq_ref, wk_ref, wv_ref, q_ref, kv_ref):
    x = x_ref[...]
    q_ref[...] = jnp.dot(x, wq_ref[...], preferred_element_type=jnp.float32)
    k = jnp.dot(x, wk_ref[...], preferred_element_type=jnp.float32)
    v = jnp.dot(x, wv_ref[...], preferred_element_type=jnp.float32)
    kv_ref[...] = jnp.concatenate([k, v], axis=1)


def _node_qkv(x, wq, wk, wv, blk=2000):
    n = x.shape[0]
    grid = n // blk
    return pl.pallas_call(
        _qkv_body,
        grid=(grid,),
        in_specs=[
            pl.BlockSpec((blk, F), lambda i: (i, 0)),
            pl.BlockSpec((F, F), lambda i: (0, 0)),
            pl.BlockSpec((F, F), lambda i: (0, 0)),
            pl.BlockSpec((F, F), lambda i: (0, 0)),
        ],
        out_specs=[
            pl.BlockSpec((blk, F), lambda i: (i, 0)),
            pl.BlockSpec((blk, 2 * F), lambda i: (i, 0)),
        ],
        out_shape=[
            jax.ShapeDtypeStruct((n, F), jnp.float32),
            jax.ShapeDtypeStruct((n, 2 * F), jnp.float32),
        ],
    )(x, wq, wk, wv)


def _edge_feat_body(qd_ref, kvs_ref, rbfc_ref, wr_ref, mh_ref, mht_ref, m_ref):
    w = jnp.dot(rbfc_ref[...], wr_ref[...], preferred_element_type=jnp.float32)
    kvs = kvs_ref[...]
    k = kvs[:, :F]
    v = kvs[:, F:]
    p = qd_ref[...] * w * k
    alpha = jnp.dot(p, mh_ref[...], preferred_element_type=jnp.float32)
    m_ref[...] = jnp.dot(alpha, mht_ref[...], preferred_element_type=jnp.float32) * v


def _edge_feat(qd, kvs, rbfc, wr, blk=2000):
    e = qd.shape[0]
    grid = e // blk
    return pl.pallas_call(
        _edge_feat_body,
        grid=(grid,),
        in_specs=[
            pl.BlockSpec((blk, F), lambda i: (i, 0)),
            pl.BlockSpec((blk, 2 * F), lambda i: (i, 0)),
            pl.BlockSpec((blk, NUM_RBF), lambda i: (i, 0)),
            pl.BlockSpec((NUM_RBF, F), lambda i: (0, 0)),
            pl.BlockSpec((F, H), lambda i: (0, 0)),
            pl.BlockSpec((H, F), lambda i: (0, 0)),
        ],
        out_specs=pl.BlockSpec((blk, F), lambda i: (i, 0)),
        out_shape=jax.ShapeDtypeStruct((e, F), jnp.float32),
    )(qd, kvs, rbfc, wr, jnp.asarray(_MH), jnp.asarray(_MHT))


def _edge_ev_body(qed_ref, kes_ref, rbfc_ref, sph_ref, wre_ref, me_ref, rep2_ref, o_ref):
    we = jnp.dot(rbfc_ref[...], wre_ref[...], preferred_element_type=jnp.float32)
    pe = qed_ref[...] * we * kes_ref[...]
    ae = jnp.dot(pe, me_ref[...], preferred_element_type=jnp.float32)
    rep = jnp.dot(ae, rep2_ref[...], preferred_element_type=jnp.float32)
    o_ref[...] = jnp.concatenate(
        [rep * sph_ref[...], jnp.zeros((rep.shape[0], F - 8), jnp.float32)], axis=1)


def _edge_ev(qed, kes, rbfc, sph, wre, blk=2000):
    e = qed.shape[0]
    grid = e // blk
    return pl.pallas_call(
        _edge_ev_body,
        grid=(grid,),
        in_specs=[
            pl.BlockSpec((blk, F), lambda i: (i, 0)),
            pl.BlockSpec((blk, F), lambda i: (i, 0)),
            pl.BlockSpec((blk, NUM_RBF), lambda i: (i, 0)),
            pl.BlockSpec((blk, 8), lambda i: (i, 0)),
            pl.BlockSpec((NUM_RBF, F), lambda i: (0, 0)),
            pl.BlockSpec((F, ND), lambda i: (0, 0)),
            pl.BlockSpec((ND, 8), lambda i: (0, 0)),
        ],
        out_specs=pl.BlockSpec((blk, F), lambda i: (i, 0)),
        out_shape=jax.ShapeDtypeStruct((e, F), jnp.float32),
    )(qed, kes, rbfc, sph, wre, jnp.asarray(_ME), jnp.asarray(_REP2))


def _nb_body(x_ref, agg_ref, wqe_ref, wke_ref, xn_ref, qe_ref, ke_ref):
    xn = x_ref[...] + (agg_ref[0] + agg_ref[1]) * (1.0 / AVG_NEI)
    xn_ref[...] = xn
    qe_ref[...] = jnp.dot(xn, wqe_ref[...], preferred_element_type=jnp.float32)
    ke_ref[...] = jnp.dot(xn, wke_ref[...], preferred_element_type=jnp.float32)


def _node_nb(x, agg, wqe, wke, blk=2000):
    n = x.shape[0]
    grid = n // blk
    return pl.pallas_call(
        _nb_body,
        grid=(grid,),
        in_specs=[
            pl.BlockSpec((blk, F), lambda i: (i, 0)),
            pl.BlockSpec((NC, blk, F), lambda i: (0, i, 0)),
            pl.BlockSpec((F, F), lambda i: (0, 0)),
            pl.BlockSpec((F, F), lambda i: (0, 0)),
        ],
        out_specs=[
            pl.BlockSpec((blk, F), lambda i: (i, 0)),
            pl.BlockSpec((blk, F), lambda i: (i, 0)),
            pl.BlockSpec((blk, F), lambda i: (i, 0)),
        ],
        out_shape=[
            jax.ShapeDtypeStruct((n, F), jnp.float32),
            jax.ShapeDtypeStruct((n, F), jnp.float32),
            jax.ShapeDtypeStruct((n, F), jnp.float32),
        ],
    )(x, agg, wqe, wke)


def _nc_body(x_ref, ev_ref, evagg_ref, w1a_ref, w1b_ref, b1_ref, w2a_ref,
             w2b_ref, b2a_ref, b2b_ref, rep2_ref, rep2t_ref, xo_ref, evo_ref):
    x = x_ref[...]
    ev1 = ev_ref[...] + (evagg_ref[0][:, :8] + evagg_ref[1][:, :8]) * (1.0 / AVG_NEI)
    d2 = jnp.dot(ev1 * ev1, rep2t_ref[...], preferred_element_type=jnp.float32)
    h = jnp.dot(x, w1a_ref[...], preferred_element_type=jnp.float32)
    h = h + jnp.dot(d2, w1b_ref[...], preferred_element_type=jnp.float32)
    h = jax.nn.silu(h + b1_ref[...])
    xo_ref[...] = x + jnp.dot(h, w2a_ref[...], preferred_element_type=jnp.float32) + b2a_ref[...]
    dsc = jnp.dot(h, w2b_ref[...], preferred_element_type=jnp.float32) + b2b_ref[...]
    evrep = jnp.dot(dsc, rep2_ref[...], preferred_element_type=jnp.float32)
    evo_ref[...] = ev1 + evrep * ev1


def _node_nc(x, ev, evagg, w1a, w1b, b1l, w2a, w2b, b2a, b2b, blk=2000):
    n = x.shape[0]
    grid = n // blk
    return pl.pallas_call(
        _nc_body,
        grid=(grid,),
        in_specs=[
            pl.BlockSpec((blk, F), lambda i: (i, 0)),
            pl.BlockSpec((blk, 8), lambda i: (i, 0)),
            pl.BlockSpec((NC, blk, F), lambda i: (0, i, 0)),
            pl.BlockSpec((F, F), lambda i: (0, 0)),
            pl.BlockSpec((ND, F), lambda i: (0, 0)),
            pl.BlockSpec((F,), lambda i: (0,)),
            pl.BlockSpec((F, F), lambda i: (0, 0)),
            pl.BlockSpec((F, ND), lambda i: (0, 0)),
            pl.BlockSpec((F,), lambda i: (0,)),
            pl.BlockSpec((ND,), lambda i: (0,)),
            pl.BlockSpec((ND, 8), lambda i: (0, 0)),
            pl.BlockSpec((8, ND), lambda i: (0, 0)),
        ],
        out_specs=[
            pl.BlockSpec((blk, F), lambda i: (i, 0)),
            pl.BlockSpec((blk, 8), lambda i: (i, 0)),
        ],
        out_shape=[
            jax.ShapeDtypeStruct((n, F), jnp.float32),
            jax.ShapeDtypeStruct((n, 8), jnp.float32),
        ],
    )(x, ev, evagg, w1a, w1b, b1l, w2a, w2b, b2a, b2b,
      jnp.asarray(_REP2), jnp.asarray(_REP2T))


def _readout_body(x_ref, wh1_ref, bh1_ref, wh2_ref, bh2_ref, out_ref):
    i = pl.program_id(0)
    h = jax.nn.silu(jnp.dot(x_ref[...], wh1_ref[...], preferred_element_type=jnp.float32) + bh1_ref[...])
    e = jnp.dot(h, wh2_ref[...], preferred_element_type=jnp.float32) + bh2_ref[...]
    s = jnp.sum(e).reshape(1, 1)

    @pl.when(i == 0)
    def _():
        out_ref[...] = jnp.zeros_like(out_ref)

    out_ref[...] += s


def _readout(x, wh1, bh1, wh2, bh2, blk=2000):
    n = x.shape[0]
    grid = n // blk
    out = pl.pallas_call(
        _readout_body,
        grid=(grid,),
        in_specs=[
            pl.BlockSpec((blk, F), lambda i: (i, 0)),
            pl.BlockSpec((F, F), lambda i: (0, 0)),
            pl.BlockSpec((F,), lambda i: (0,)),
            pl.BlockSpec((F, 1), lambda i: (0, 0)),
            pl.BlockSpec((1,), lambda i: (0,)),
        ],
        out_specs=pl.BlockSpec((1, 1), lambda i: (0, 0)),
        out_shape=jax.ShapeDtypeStruct((1, 1), jnp.float32),
    )(x, wh1, bh1, wh2, bh2)
    return out.reshape(1)


def kernel(positions, atomic_numbers, edge_index, emb, Wq, Wk, Wv, Wr, Wqe,
           Wke, Wre, W1, b1, W2, b2, Wh1, bh1, Wh2, bh2):
    n = positions.shape[0]
    src = edge_index[0]
    dst = edge_index[1]

    pos4 = jnp.pad(positions, ((0, 0), (0, 1)))
    dvec = _sc_edge_vec(pos4, src, dst)
    rbfc, sph = _geom(dvec)

    an = atomic_numbers.astype(jnp.int32)
    n_pad = ((n + NW * 8 - 1) // (NW * 8)) * (NW * 8)
    an_pad = jnp.pad(an, (0, n_pad - n))
    x = _sc_gather(emb, an_pad, F)[:n]

    ev = jnp.zeros((n, 8), jnp.float32)
    num_layers = Wq.shape[0]
    for l in range(num_layers):
        q, kv = _node_qkv(x, Wq[l], Wk[l], Wv[l])
        qd = _sc_gather(q, dst, F)
        kvs = _sc_gather(kv, src, 2 * F)
        m = _edge_feat(qd, kvs, rbfc, Wr[l])
        agg = _sc_scatter_add(m, dst, n, F)
        x, qe, ke = _node_nb(x, agg, Wqe[l], Wke[l])
        qed = _sc_gather(qe, dst, F)
        kes = _sc_gather(ke, src, F)
        rv = _edge_ev(qed, kes, rbfc, sph, Wre[l])
        evagg = _sc_scatter_add(rv, dst, n, F)
        x, ev = _node_nc(x, ev, evagg, W1[l][:F], W1[l][F:], b1[l],
                         W2[l][:, :F], W2[l][:, F:], b2[l][:F], b2[l][F:])
    return _readout(x, Wh1, bh1, Wh2, bh2)
